# baseline (speedup 1.0000x reference)
import functools

import jax
import jax.numpy as jnp
from jax import lax
from jax.experimental import pallas as pl
from jax.experimental.pallas import tpu as pltpu
from jax.experimental.pallas import tpu_sc as plsc

_N = 10000
_E = 320000
_D = 128
_H = 128
_OUT = 2
_G = 64

_NC = 2
_NS = 16
_NW = _NC * _NS
_CHUNK = 128
_CPW = 80
_EPW = _CPW * _CHUNK
_EPAD = _NW * _EPW
_ACC_ROWS = 10240
_ZPT = _ACC_ROWS // _NS
_OPT = _N // _NS


def _sc_mesh():
  return plsc.VectorSubcoreMesh(core_axis_name="c", subcore_axis_name="s",
                                num_cores=_NC, num_subcores=_NS)


_GRP = 32
_CPS = _NW * _CPW // _NS
_DEGW = 8


def _emit_partial(acc, out, c, s):
  o0 = s * 624
  for k in range(3):
    sl = pl.ds(o0 + k * 208, 208)
    pltpu.sync_copy(acc.at[sl], out.at[c].at[sl])
  @pl.when(s == 0)
  def _tail():
    sl = pl.ds(_NS * 624, _N - _NS * 624)
    pltpu.sync_copy(acc.at[sl], out.at[c].at[sl])


def _zero_rows(bufs, i):
  for l in range(_H // 16):
    bufs[0, i, pl.ds(l * 16, 16)] = jnp.zeros((16,), jnp.float32)


def _make_sc_pass():
  out_type = [jax.ShapeDtypeStruct((_NC, _N, _H), jnp.float32)]
  scratch = [
      pltpu.VMEM((_GRP, _CHUNK), jnp.int32),
      pltpu.VMEM((_GRP, _CHUNK), jnp.int32),
      pltpu.VMEM((2, _CHUNK, _H), jnp.float32),
      pltpu.VMEM_SHARED((_ACC_ROWS, _H), jnp.float32),
      pltpu.SemaphoreType.DMA,
      pltpu.SemaphoreType.DMA,
  ]

  def body(y, srcm, dstm, out, dst_g, src_g, bufs, acc, ssem, gsem):
    c = lax.axis_index("c")
    s = lax.axis_index("s")

    def fill_rows(i, _):
      _zero_rows(bufs, i)
      return 0
    lax.fori_loop(0, _CHUNK, fill_rows, 0)
    @pl.when(c == 1)
    def _fill_ones():
      def ones_rows(i, _):
        for l in range(_H // 16):
          bufs[1, i, pl.ds(l * 16, 16)] = jnp.ones((16,), jnp.float32)
        return 0
      lax.fori_loop(0, _CHUNK, ones_rows, 0)

    z0 = s * _ZPT
    for k in range(_ZPT // _CHUNK):
      pltpu.sync_copy(bufs.at[0], acc.at[pl.ds(z0 + k * _CHUNK, _CHUNK)])
    plsc.subcore_barrier()

    for grp in range(_CPS // _GRP):
      g0 = s * _CPS + grp * _GRP
      pltpu.sync_copy(dstm.at[pl.ds(g0, _GRP)], dst_g)

      @pl.when(c == 0)
      def _messages():
        pltpu.sync_copy(srcm.at[pl.ds(g0, _GRP)], src_g)
        pltpu.async_copy(y.at[src_g.at[0]], bufs.at[0], gsem)

        def step(j, _):
          par = lax.rem(j, 2)
          pltpu.make_async_copy(y.at[src_g.at[j]], bufs.at[par], gsem).wait()
          @pl.when(j >= 1)
          def _wait_prev_scatter():
            pltpu.make_async_copy(bufs.at[1 - par],
                                  acc.at[dst_g.at[j - 1]], ssem).wait()
          @pl.when(j < _GRP - 1)
          def _fire_next_gather():
            pltpu.async_copy(y.at[src_g.at[j + 1]], bufs.at[1 - par], gsem)
          pltpu.async_copy(bufs.at[par], acc.at[dst_g.at[j]], ssem, add=True)
          return 0
        lax.fori_loop(0, _GRP, step, 0)
        pltpu.make_async_copy(bufs.at[(_GRP - 1) % 2],
                              acc.at[dst_g.at[_GRP - 1]], ssem).wait()

      @pl.when(c == 1)
      def _degrees():
        def step(j, _):
          pltpu.async_copy(bufs.at[1], acc.at[dst_g.at[j]], ssem, add=True)
          @pl.when(j >= _DEGW)
          def _wait_old():
            pltpu.make_async_copy(bufs.at[1], acc.at[dst_g.at[j - _DEGW]],
                                  ssem).wait()
          return 0
        lax.fori_loop(0, _GRP, step, 0)
        for t in range(_DEGW):
          pltpu.make_async_copy(bufs.at[1], acc.at[dst_g.at[_GRP - _DEGW + t]],
                                ssem).wait()

    plsc.subcore_barrier()
    _emit_partial(acc, out, c, s)

  return pl.kernel(body, out_type=out_type, mesh=_sc_mesh(),
                   scratch_types=scratch)


def _tc_init(g0_r, g1_r, g2_r, wl_r, wr_r, bl_r, y_r, z_r):
  x = jnp.concatenate([g0_r[...], g1_r[...], g2_r[...]], axis=1)
  y_r[...] = jnp.dot(x, wl_r[...], preferred_element_type=jnp.float32)
  z_r[...] = jnp.dot(x, wr_r[...],
                     preferred_element_type=jnp.float32) + bl_r[...]


def _combine_bn_relu(part_r, z_r, gamma_r, beta_r):
  ssum = part_r[0]
  deg = part_r[1][:, 0:1]
  h = ssum / jnp.maximum(deg, 1.0) + z_r[...]
  mu = jnp.mean(h, axis=0, keepdims=True)
  dv = h - mu
  var = jnp.mean(dv * dv, axis=0, keepdims=True)
  xn = dv * lax.rsqrt(var + 1e-5) * gamma_r[...] + beta_r[...]
  return jnp.maximum(xn, 0.0)


def _tc_combine(part_r, z_r, gamma_r, beta_r, wl_r, wr_r, bl_r,
                y_r, z2_r):
  xn = _combine_bn_relu(part_r, z_r, gamma_r, beta_r)
  y_r[...] = jnp.dot(xn, wl_r[...], preferred_element_type=jnp.float32)
  z2_r[...] = jnp.dot(xn, wr_r[...],
                      preferred_element_type=jnp.float32) + bl_r[...]


def _tc_final(part_r, z_r, gamma_r, beta_r, batch_r, fcw_r, fcb_r,
              w1_r, b1_r, w2_r, b2_r, o_r):
  xn = _combine_bn_relu(part_r, z_r, gamma_r, beta_r)
  b = batch_r[...]
  gid = lax.broadcasted_iota(jnp.int32, (_N, _G), 1)
  oh = (b == gid).astype(jnp.float32)
  pooled = lax.dot_general(oh, xn, (((0,), (0,)), ((), ())),
                           preferred_element_type=jnp.float32)
  cnt = jnp.sum(oh, axis=0)[:, None]
  p = pooled / jnp.maximum(cnt, 1.0)
  r = jnp.maximum(
      jnp.dot(p, fcw_r[...], preferred_element_type=jnp.float32)
      + fcb_r[...], 0.0)
  r = jnp.maximum(
      jnp.dot(r, w1_r[...], preferred_element_type=jnp.float32)
      + b1_r[...], 0.0)
  o_r[...] = jnp.dot(r, w2_r[...],
                     preferred_element_type=jnp.float32) + b2_r[...]


_f32 = jnp.float32


def _nh(k=2):
  return [jax.ShapeDtypeStruct((_N, _H), _f32) for _ in range(k)]


_make_sc_pass = functools.lru_cache(maxsize=None)(_make_sc_pass)
_init_call = pl.pallas_call(_tc_init, out_shape=_nh())
_combine_call = pl.pallas_call(_tc_combine, out_shape=_nh())
_final_call = pl.pallas_call(
    _tc_final, out_shape=jax.ShapeDtypeStruct((_G, 128), _f32))


def kernel(g0, g1, g2, edge_index, batch, Wl0, bl0, Wr0, Wl, bl, Wr,
           bn_gamma, bn_beta, fc_W, fc_b, mlp_W1, mlp_b1, mlp_W2, mlp_b2):
  src = edge_index[0].astype(jnp.int32)
  dst = edge_index[1].astype(jnp.int32)
  pad = _EPAD - _E
  srcm = jnp.concatenate([src, jnp.zeros((pad,), jnp.int32)])
  srcm = srcm.reshape(_NW * _CPW, _CHUNK)
  dstm = jnp.concatenate([dst, jnp.full((pad,), _N, jnp.int32)])
  dstm = dstm.reshape(_NW * _CPW, _CHUNK)

  y, z = _init_call(g0, g1, g2, Wl0, Wr0, bl0.reshape(1, _H))
  (part,) = _make_sc_pass()(y, srcm, dstm)
  for i in range(3):
    y, z = _combine_call(part, z,
                         bn_gamma[i].reshape(1, _H), bn_beta[i].reshape(1, _H),
                         Wl[i], Wr[i], bl[i].reshape(1, _H))
    (part,) = _make_sc_pass()(y, srcm, dstm)

  w2p = jnp.pad(mlp_W2, ((0, 0), (0, 128 - _OUT)))
  b2p = jnp.pad(mlp_b2, ((0, 128 - _OUT))).reshape(1, 128)
  out = _final_call(part, z,
                    bn_gamma[3].reshape(1, _H), bn_beta[3].reshape(1, _H),
                    batch.astype(jnp.int32).reshape(_N, 1),
                    fc_W, fc_b.reshape(1, _H),
                    mlp_W1, mlp_b1.reshape(1, _H), w2p, b2p)
  return out[:, :_OUT]


# speedup vs baseline: 3.5267x; 3.5267x over previous
import functools

import jax
import jax.numpy as jnp
from jax import lax
from jax.experimental import pallas as pl
from jax.experimental.pallas import tpu as pltpu
from jax.experimental.pallas import tpu_sc as plsc

_N = 10000
_E = 320000
_D = 128
_H = 128
_OUT = 2
_G = 64

_NC = 2
_NS = 16
_NW = _NC * _NS
_CHUNK = 128
_CPW = 80
_EPW = _CPW * _CHUNK
_EPAD = _NW * _EPW
_ACC_ROWS = 10240
_ZPT = _ACC_ROWS // _NS
_OPT = _N // _NS


def _sc_mesh():
  return plsc.VectorSubcoreMesh(core_axis_name="c", subcore_axis_name="s",
                                num_cores=_NC, num_subcores=_NS)


_GRP = 40
_DEGW = 8


def _emit_partial(acc, out, c, s):
  o0 = s * 624
  for k in range(3):
    sl = pl.ds(o0 + k * 208, 208)
    pltpu.sync_copy(acc.at[sl], out.at[c].at[sl])
  @pl.when(s == 0)
  def _tail():
    sl = pl.ds(_NS * 624, _N - _NS * 624)
    pltpu.sync_copy(acc.at[sl], out.at[c].at[sl])


def _zero_rows(bufs, i):
  for l in range(_H // 16):
    bufs[0, i, pl.ds(l * 16, 16)] = jnp.zeros((16,), jnp.float32)


def _make_sc_msg():
  out_type = [jax.ShapeDtypeStruct((_NC, _N, _H), jnp.float32)]
  scratch = [
      pltpu.VMEM((_GRP, _CHUNK), jnp.int32),
      pltpu.VMEM((_GRP, _CHUNK), jnp.int32),
      pltpu.VMEM((2, _CHUNK, _H), jnp.float32),
      pltpu.VMEM_SHARED((_ACC_ROWS, _H), jnp.float32),
      pltpu.SemaphoreType.DMA,
      pltpu.SemaphoreType.DMA,
  ]

  def body(y, srcm, dstm, out, dst_g, src_g, bufs, acc, ssem, gsem):
    c = lax.axis_index("c")
    s = lax.axis_index("s")
    wid = s * _NC + c

    def fill_rows(i, _):
      _zero_rows(bufs, i)
      return 0
    lax.fori_loop(0, _CHUNK, fill_rows, 0)

    z0 = s * _ZPT
    for k in range(_ZPT // _CHUNK):
      pltpu.sync_copy(bufs.at[0], acc.at[pl.ds(z0 + k * _CHUNK, _CHUNK)])
    plsc.subcore_barrier()

    for grp in range(_CPW // _GRP):
      g0 = wid * _CPW + grp * _GRP
      pltpu.sync_copy(dstm.at[pl.ds(g0, _GRP)], dst_g)
      pltpu.sync_copy(srcm.at[pl.ds(g0, _GRP)], src_g)
      pltpu.async_copy(y.at[src_g.at[0]], bufs.at[0], gsem)

      def step(j, _):
        par = lax.rem(j, 2)
        pltpu.make_async_copy(y.at[src_g.at[j]], bufs.at[par], gsem).wait()
        @pl.when(j >= 1)
        def _wait_prev_scatter():
          pltpu.make_async_copy(bufs.at[1 - par],
                                acc.at[dst_g.at[j - 1]], ssem).wait()
        @pl.when(j < _GRP - 1)
        def _fire_next_gather():
          pltpu.async_copy(y.at[src_g.at[j + 1]], bufs.at[1 - par], gsem)
        pltpu.async_copy(bufs.at[par], acc.at[dst_g.at[j]], ssem, add=True)
        return 0
      lax.fori_loop(0, _GRP, step, 0)
      pltpu.make_async_copy(bufs.at[(_GRP - 1) % 2],
                            acc.at[dst_g.at[_GRP - 1]], ssem).wait()

    plsc.subcore_barrier()
    _emit_partial(acc, out, c, s)

  return pl.kernel(body, out_type=out_type, mesh=_sc_mesh(),
                   scratch_types=scratch)


def _make_sc_deg():
  out_type = [jax.ShapeDtypeStruct((_NC, _N, _H), jnp.float32)]
  scratch = [
      pltpu.VMEM((_GRP, _CHUNK), jnp.int32),
      pltpu.VMEM((2, _CHUNK, _H), jnp.float32),
      pltpu.VMEM_SHARED((_ACC_ROWS, _H), jnp.float32),
      pltpu.SemaphoreType.DMA,
  ]

  def body(dstm, out, dst_g, bufs, acc, ssem):
    c = lax.axis_index("c")
    s = lax.axis_index("s")
    wid = s * _NC + c

    def fill_rows(i, _):
      _zero_rows(bufs, i)
      for l in range(_H // 16):
        bufs[1, i, pl.ds(l * 16, 16)] = jnp.ones((16,), jnp.float32)
      return 0
    lax.fori_loop(0, _CHUNK, fill_rows, 0)

    z0 = s * _ZPT
    for k in range(_ZPT // _CHUNK):
      pltpu.sync_copy(bufs.at[0], acc.at[pl.ds(z0 + k * _CHUNK, _CHUNK)])
    plsc.subcore_barrier()

    for grp in range(_CPW // _GRP):
      g0 = wid * _CPW + grp * _GRP
      pltpu.sync_copy(dstm.at[pl.ds(g0, _GRP)], dst_g)

      def step(j, _):
        pltpu.async_copy(bufs.at[1], acc.at[dst_g.at[j]], ssem, add=True)
        @pl.when(j >= _DEGW)
        def _wait_old():
          pltpu.make_async_copy(bufs.at[1], acc.at[dst_g.at[j - _DEGW]],
                                ssem).wait()
        return 0
      lax.fori_loop(0, _GRP, step, 0)
      for t in range(_DEGW):
        pltpu.make_async_copy(bufs.at[1], acc.at[dst_g.at[_GRP - _DEGW + t]],
                              ssem).wait()

    plsc.subcore_barrier()
    _emit_partial(acc, out, c, s)

  return pl.kernel(body, out_type=out_type, mesh=_sc_mesh(),
                   scratch_types=scratch)


def _tc_init(g0_r, g1_r, g2_r, wl_r, wr_r, bl_r, y_r, z_r):
  x = jnp.concatenate([g0_r[...], g1_r[...], g2_r[...]], axis=1)
  y_r[...] = jnp.dot(x, wl_r[...], preferred_element_type=jnp.float32)
  z_r[...] = jnp.dot(x, wr_r[...],
                     preferred_element_type=jnp.float32) + bl_r[...]


def _combine_bn_relu(part_r, degp_r, z_r, gamma_r, beta_r):
  ssum = part_r[0] + part_r[1]
  deg = (degp_r[0] + degp_r[1])[:, 0:1]
  h = ssum / jnp.maximum(deg, 1.0) + z_r[...]
  mu = jnp.mean(h, axis=0, keepdims=True)
  dv = h - mu
  var = jnp.mean(dv * dv, axis=0, keepdims=True)
  xn = dv * lax.rsqrt(var + 1e-5) * gamma_r[...] + beta_r[...]
  return jnp.maximum(xn, 0.0)


def _tc_combine(part_r, degp_r, z_r, gamma_r, beta_r, wl_r, wr_r, bl_r,
                y_r, z2_r):
  xn = _combine_bn_relu(part_r, degp_r, z_r, gamma_r, beta_r)
  y_r[...] = jnp.dot(xn, wl_r[...], preferred_element_type=jnp.float32)
  z2_r[...] = jnp.dot(xn, wr_r[...],
                      preferred_element_type=jnp.float32) + bl_r[...]


def _tc_final(part_r, degp_r, z_r, gamma_r, beta_r, batch_r, fcw_r, fcb_r,
              w1_r, b1_r, w2_r, b2_r, o_r):
  xn = _combine_bn_relu(part_r, degp_r, z_r, gamma_r, beta_r)
  b = batch_r[...]
  gid = lax.broadcasted_iota(jnp.int32, (_N, _G), 1)
  oh = (b == gid).astype(jnp.float32)
  pooled = lax.dot_general(oh, xn, (((0,), (0,)), ((), ())),
                           preferred_element_type=jnp.float32)
  cnt = jnp.sum(oh, axis=0)[:, None]
  p = pooled / jnp.maximum(cnt, 1.0)
  r = jnp.maximum(
      jnp.dot(p, fcw_r[...], preferred_element_type=jnp.float32)
      + fcb_r[...], 0.0)
  r = jnp.maximum(
      jnp.dot(r, w1_r[...], preferred_element_type=jnp.float32)
      + b1_r[...], 0.0)
  o_r[...] = jnp.dot(r, w2_r[...],
                     preferred_element_type=jnp.float32) + b2_r[...]


_f32 = jnp.float32


def _nh(k=2):
  return [jax.ShapeDtypeStruct((_N, _H), _f32) for _ in range(k)]


_make_sc_msg = functools.lru_cache(maxsize=None)(_make_sc_msg)
_make_sc_deg = functools.lru_cache(maxsize=None)(_make_sc_deg)
_init_call = pl.pallas_call(_tc_init, out_shape=_nh())
_combine_call = pl.pallas_call(_tc_combine, out_shape=_nh())
_final_call = pl.pallas_call(
    _tc_final, out_shape=jax.ShapeDtypeStruct((_G, 128), _f32))


def kernel(g0, g1, g2, edge_index, batch, Wl0, bl0, Wr0, Wl, bl, Wr,
           bn_gamma, bn_beta, fc_W, fc_b, mlp_W1, mlp_b1, mlp_W2, mlp_b2):
  src = edge_index[0].astype(jnp.int32)
  dst = edge_index[1].astype(jnp.int32)
  pad = _EPAD - _E
  k = jnp.arange(pad, dtype=jnp.int32)
  srcm = jnp.concatenate([src, k % 8192]).reshape(_NW * _CPW, _CHUNK)
  dstm = jnp.concatenate([dst, _N + (k % (_ACC_ROWS - _N))])
  dstm = dstm.reshape(_NW * _CPW, _CHUNK)

  y, z = _init_call(g0, g1, g2, Wl0, Wr0, bl0.reshape(1, _H))
  (degp,) = _make_sc_deg()(dstm)
  (part,) = _make_sc_msg()(y, srcm, dstm)
  for i in range(3):
    y, z = _combine_call(part, degp, z,
                         bn_gamma[i].reshape(1, _H), bn_beta[i].reshape(1, _H),
                         Wl[i], Wr[i], bl[i].reshape(1, _H))
    (part,) = _make_sc_msg()(y, srcm, dstm)

  w2p = jnp.pad(mlp_W2, ((0, 0), (0, 128 - _OUT)))
  b2p = jnp.pad(mlp_b2, ((0, 128 - _OUT))).reshape(1, 128)
  out = _final_call(part, degp, z,
                    bn_gamma[3].reshape(1, _H), bn_beta[3].reshape(1, _H),
                    batch.astype(jnp.int32).reshape(_N, 1),
                    fc_W, fc_b.reshape(1, _H),
                    mlp_W1, mlp_b1.reshape(1, _H), w2p, b2p)
  return out[:, :_OUT]


# speedup vs baseline: 3.6087x; 1.0232x over previous
import functools

import jax
import jax.numpy as jnp
from jax import lax
from jax.experimental import pallas as pl
from jax.experimental.pallas import tpu as pltpu
from jax.experimental.pallas import tpu_sc as plsc

_N = 10000
_E = 320000
_D = 128
_H = 128
_OUT = 2
_G = 64

_NC = 2
_NS = 16
_NW = _NC * _NS
_CHUNK = 128
_CPW = 80
_EPW = _CPW * _CHUNK
_EPAD = _NW * _EPW
_ACC_ROWS = 10240
_ZPT = _ACC_ROWS // _NS
_OPT = _N // _NS


def _sc_mesh():
  return plsc.VectorSubcoreMesh(core_axis_name="c", subcore_axis_name="s",
                                num_cores=_NC, num_subcores=_NS)


_GRP = 40
_DEGW = 8


def _emit_partial(acc, out, c, s):
  o0 = s * 624
  for k in range(3):
    sl = pl.ds(o0 + k * 208, 208)
    pltpu.sync_copy(acc.at[sl], out.at[c].at[sl])
  @pl.when(s == 0)
  def _tail():
    sl = pl.ds(_NS * 624, _N - _NS * 624)
    pltpu.sync_copy(acc.at[sl], out.at[c].at[sl])


def _zero_rows(bufs, i):
  for l in range(_H // 16):
    bufs[0, i, pl.ds(l * 16, 16)] = jnp.zeros((16,), jnp.float32)


def _make_sc_msg():
  out_type = [jax.ShapeDtypeStruct((_NC, _N, _H), jnp.float32)]
  scratch = [
      pltpu.VMEM((_GRP, _CHUNK), jnp.int32),
      pltpu.VMEM((_GRP, _CHUNK), jnp.int32),
      pltpu.VMEM((2, _CHUNK, _H), jnp.float32),
      pltpu.VMEM_SHARED((_ACC_ROWS, _H), jnp.float32),
      pltpu.SemaphoreType.DMA,
      pltpu.SemaphoreType.DMA,
      pltpu.SemaphoreType.DMA,
  ]
  _HC = _CHUNK // 2

  def body(y, srcm, dstm, out, dst_g, src_g, bufs, acc, ssem, gsem, gsem2):
    c = lax.axis_index("c")
    s = lax.axis_index("s")
    wid = s * _NC + c

    def fill_rows(i, _):
      _zero_rows(bufs, i)
      return 0
    lax.fori_loop(0, _CHUNK, fill_rows, 0)

    z0 = s * _ZPT
    for k in range(_ZPT // _CHUNK):
      pltpu.sync_copy(bufs.at[0], acc.at[pl.ds(z0 + k * _CHUNK, _CHUNK)])
    plsc.subcore_barrier()

    for grp in range(_CPW // _GRP):
      g0 = wid * _CPW + grp * _GRP
      pltpu.sync_copy(dstm.at[pl.ds(g0, _GRP)], dst_g)
      pltpu.sync_copy(srcm.at[pl.ds(g0, _GRP)], src_g)

      def fire_gather(j, par):
        pltpu.async_copy(y.at[src_g.at[j, pl.ds(0, _HC)]],
                         bufs.at[par, pl.ds(0, _HC)], gsem)
        pltpu.async_copy(y.at[src_g.at[j, pl.ds(_HC, _HC)]],
                         bufs.at[par, pl.ds(_HC, _HC)], gsem2)

      def wait_gather(j, par):
        pltpu.make_async_copy(y.at[src_g.at[j, pl.ds(0, _HC)]],
                              bufs.at[par, pl.ds(0, _HC)], gsem).wait()
        pltpu.make_async_copy(y.at[src_g.at[j, pl.ds(_HC, _HC)]],
                              bufs.at[par, pl.ds(_HC, _HC)], gsem2).wait()

      fire_gather(0, 0)

      def step(j, _):
        par = lax.rem(j, 2)
        wait_gather(j, par)
        @pl.when(j >= 1)
        def _wait_prev_scatter():
          pltpu.make_async_copy(bufs.at[1 - par],
                                acc.at[dst_g.at[j - 1]], ssem).wait()
        @pl.when(j < _GRP - 1)
        def _fire_next_gather():
          fire_gather(j + 1, 1 - par)
        pltpu.async_copy(bufs.at[par], acc.at[dst_g.at[j]], ssem, add=True)
        return 0
      lax.fori_loop(0, _GRP, step, 0)
      pltpu.make_async_copy(bufs.at[(_GRP - 1) % 2],
                            acc.at[dst_g.at[_GRP - 1]], ssem).wait()

    plsc.subcore_barrier()
    _emit_partial(acc, out, c, s)

  return pl.kernel(body, out_type=out_type, mesh=_sc_mesh(),
                   scratch_types=scratch)


def _make_sc_deg():
  out_type = [jax.ShapeDtypeStruct((_NC, _N, _H), jnp.float32)]
  scratch = [
      pltpu.VMEM((_GRP, _CHUNK), jnp.int32),
      pltpu.VMEM((2, _CHUNK, _H), jnp.float32),
      pltpu.VMEM_SHARED((_ACC_ROWS, _H), jnp.float32),
      pltpu.SemaphoreType.DMA,
  ]

  def body(dstm, out, dst_g, bufs, acc, ssem):
    c = lax.axis_index("c")
    s = lax.axis_index("s")
    wid = s * _NC + c

    def fill_rows(i, _):
      _zero_rows(bufs, i)
      for l in range(_H // 16):
        bufs[1, i, pl.ds(l * 16, 16)] = jnp.ones((16,), jnp.float32)
      return 0
    lax.fori_loop(0, _CHUNK, fill_rows, 0)

    z0 = s * _ZPT
    for k in range(_ZPT // _CHUNK):
      pltpu.sync_copy(bufs.at[0], acc.at[pl.ds(z0 + k * _CHUNK, _CHUNK)])
    plsc.subcore_barrier()

    for grp in range(_CPW // _GRP):
      g0 = wid * _CPW + grp * _GRP
      pltpu.sync_copy(dstm.at[pl.ds(g0, _GRP)], dst_g)

      def step(j, _):
        pltpu.async_copy(bufs.at[1], acc.at[dst_g.at[j]], ssem, add=True)
        @pl.when(j >= _DEGW)
        def _wait_old():
          pltpu.make_async_copy(bufs.at[1], acc.at[dst_g.at[j - _DEGW]],
                                ssem).wait()
        return 0
      lax.fori_loop(0, _GRP, step, 0)
      for t in range(_DEGW):
        pltpu.make_async_copy(bufs.at[1], acc.at[dst_g.at[_GRP - _DEGW + t]],
                              ssem).wait()

    plsc.subcore_barrier()
    _emit_partial(acc, out, c, s)

  return pl.kernel(body, out_type=out_type, mesh=_sc_mesh(),
                   scratch_types=scratch)


def _tc_init(g0_r, g1_r, g2_r, wl_r, wr_r, bl_r, y_r, z_r):
  x = jnp.concatenate([g0_r[...], g1_r[...], g2_r[...]], axis=1)
  y_r[...] = jnp.dot(x, wl_r[...], preferred_element_type=jnp.float32)
  z_r[...] = jnp.dot(x, wr_r[...],
                     preferred_element_type=jnp.float32) + bl_r[...]


def _combine_bn_relu(part_r, degp_r, z_r, gamma_r, beta_r):
  ssum = part_r[0] + part_r[1]
  deg = (degp_r[0] + degp_r[1])[:, 0:1]
  h = ssum / jnp.maximum(deg, 1.0) + z_r[...]
  mu = jnp.mean(h, axis=0, keepdims=True)
  dv = h - mu
  var = jnp.mean(dv * dv, axis=0, keepdims=True)
  xn = dv * lax.rsqrt(var + 1e-5) * gamma_r[...] + beta_r[...]
  return jnp.maximum(xn, 0.0)


def _tc_combine(part_r, degp_r, z_r, gamma_r, beta_r, wl_r, wr_r, bl_r,
                y_r, z2_r):
  xn = _combine_bn_relu(part_r, degp_r, z_r, gamma_r, beta_r)
  y_r[...] = jnp.dot(xn, wl_r[...], preferred_element_type=jnp.float32)
  z2_r[...] = jnp.dot(xn, wr_r[...],
                      preferred_element_type=jnp.float32) + bl_r[...]


def _tc_final(part_r, degp_r, z_r, gamma_r, beta_r, batch_r, fcw_r, fcb_r,
              w1_r, b1_r, w2_r, b2_r, o_r):
  xn = _combine_bn_relu(part_r, degp_r, z_r, gamma_r, beta_r)
  b = batch_r[...]
  gid = lax.broadcasted_iota(jnp.int32, (_N, _G), 1)
  oh = (b == gid).astype(jnp.float32)
  pooled = lax.dot_general(oh, xn, (((0,), (0,)), ((), ())),
                           preferred_element_type=jnp.float32)
  cnt = jnp.sum(oh, axis=0)[:, None]
  p = pooled / jnp.maximum(cnt, 1.0)
  r = jnp.maximum(
      jnp.dot(p, fcw_r[...], preferred_element_type=jnp.float32)
      + fcb_r[...], 0.0)
  r = jnp.maximum(
      jnp.dot(r, w1_r[...], preferred_element_type=jnp.float32)
      + b1_r[...], 0.0)
  o_r[...] = jnp.dot(r, w2_r[...],
                     preferred_element_type=jnp.float32) + b2_r[...]


_f32 = jnp.float32


def _nh(k=2):
  return [jax.ShapeDtypeStruct((_N, _H), _f32) for _ in range(k)]


_make_sc_msg = functools.lru_cache(maxsize=None)(_make_sc_msg)
_make_sc_deg = functools.lru_cache(maxsize=None)(_make_sc_deg)
_init_call = pl.pallas_call(_tc_init, out_shape=_nh())
_combine_call = pl.pallas_call(_tc_combine, out_shape=_nh())
_final_call = pl.pallas_call(
    _tc_final, out_shape=jax.ShapeDtypeStruct((_G, 128), _f32))


def kernel(g0, g1, g2, edge_index, batch, Wl0, bl0, Wr0, Wl, bl, Wr,
           bn_gamma, bn_beta, fc_W, fc_b, mlp_W1, mlp_b1, mlp_W2, mlp_b2):
  src = edge_index[0].astype(jnp.int32)
  dst = edge_index[1].astype(jnp.int32)
  pad = _EPAD - _E
  k = jnp.arange(pad, dtype=jnp.int32)
  srcm = jnp.concatenate([src, k % 8192]).reshape(_NW * _CPW, _CHUNK)
  dstm = jnp.concatenate([dst, _N + (k % (_ACC_ROWS - _N))])
  dstm = dstm.reshape(_NW * _CPW, _CHUNK)

  y, z = _init_call(g0, g1, g2, Wl0, Wr0, bl0.reshape(1, _H))
  (degp,) = _make_sc_deg()(dstm)
  (part,) = _make_sc_msg()(y, srcm, dstm)
  for i in range(3):
    y, z = _combine_call(part, degp, z,
                         bn_gamma[i].reshape(1, _H), bn_beta[i].reshape(1, _H),
                         Wl[i], Wr[i], bl[i].reshape(1, _H))
    (part,) = _make_sc_msg()(y, srcm, dstm)

  w2p = jnp.pad(mlp_W2, ((0, 0), (0, 128 - _OUT)))
  b2p = jnp.pad(mlp_b2, ((0, 128 - _OUT))).reshape(1, 128)
  out = _final_call(part, degp, z,
                    bn_gamma[3].reshape(1, _H), bn_beta[3].reshape(1, _H),
                    batch.astype(jnp.int32).reshape(_N, 1),
                    fc_W, fc_b.reshape(1, _H),
                    mlp_W1, mlp_b1.reshape(1, _H), w2p, b2p)
  return out[:, :_OUT]
